# baseline (speedup 1.0000x reference)
import functools

import jax
import jax.numpy as jnp
import numpy as np
from jax import lax
from jax.experimental import pallas as pl
from jax.experimental.pallas import tpu as pltpu
from jax.experimental.pallas import tpu_sc as plsc

_TOTAL = 1048576 * 3 * 5
_NC = 2
_NS = 16
_NW = _NC * _NS
_PER_W = _TOTAL // _NW
_CHUNK = 40960
_ITERS = _PER_W // _CHUNK

_MASKS = np.array(
    [[1.0 if (p + l) % 5 < 3 else 0.0 for l in range(16)] for p in range(5)],
    dtype=np.float32,
).reshape(80)

_mesh = plsc.VectorSubcoreMesh(core_axis_name="c", subcore_axis_name="s")


@functools.partial(
    pl.kernel,
    mesh=_mesh,
    out_type=jax.ShapeDtypeStruct((_TOTAL,), jnp.float32),
    scratch_types=[
        pltpu.VMEM((_CHUNK,), jnp.float32),
        pltpu.VMEM((80,), jnp.float32),
    ],
)
def _sc_add(x_hbm, mask_hbm, out_hbm, buf, mbuf):
    wid = lax.axis_index("s") * _NC + lax.axis_index("c")
    base = wid * _PER_W
    pltpu.sync_copy(mask_hbm, mbuf)

    def outer(it, carry):
        off = base + it * _CHUNK
        pltpu.sync_copy(x_hbm.at[pl.ds(off, _CHUNK)], buf)

        def inner(g, c2):
            o = g * 80
            for p in range(5):
                sl = pl.ds(o + p * 16, 16)
                buf[sl] = buf[sl] + mbuf[pl.ds(p * 16, 16)]
            return c2

        lax.fori_loop(0, _CHUNK // 80, inner, 0)
        pltpu.sync_copy(buf, out_hbm.at[pl.ds(off, _CHUNK)])
        return carry

    lax.fori_loop(0, _ITERS, outer, 0)


def kernel(x):
    xf = x.reshape(_TOTAL)
    masks = jnp.asarray(_MASKS)
    return _sc_add(xf, masks).reshape(x.shape)


# speedup vs baseline: 78.3781x; 78.3781x over previous
import jax
import jax.numpy as jnp
from jax.experimental import pallas as pl

_N = 1048576
_BLOCK_N = 524288


def _add_mask_kernel(x_ref, o_ref):
    j = pl.program_id(0)

    @pl.when(j < 3)
    def _():
        o_ref[...] = x_ref[...] + 1.0

    @pl.when(j >= 3)
    def _():
        o_ref[...] = x_ref[...]


def kernel(x):
    xt = jnp.transpose(x, (2, 1, 0))
    out_t = pl.pallas_call(
        _add_mask_kernel,
        out_shape=jax.ShapeDtypeStruct((5, 3, _N), jnp.float32),
        grid=(5, _N // _BLOCK_N),
        in_specs=[pl.BlockSpec((1, 3, _BLOCK_N), lambda j, k: (j, 0, k))],
        out_specs=pl.BlockSpec((1, 3, _BLOCK_N), lambda j, k: (j, 0, k)),
    )(xt)
    return jnp.transpose(out_t, (2, 1, 0))
